# baseline (speedup 1.0000x reference)
import functools

import jax
import jax.numpy as jnp
from jax import lax
from jax.experimental import pallas as pl
from jax.experimental.pallas import tpu as pltpu
from jax.experimental.pallas import tpu_sc as plsc

EMBED_DIM = 128
HALF = EMBED_DIM // 2
ATTN_DIM = 768

_NC = 2
_NS = 16
_NW = _NC * _NS

_CHUNK = 128


def _pack_body(t_ref, out_ref):
    t = t_ref[...]
    lo = lax.bitcast_convert_type(
        t[:, :HALF].astype(jnp.bfloat16), jnp.uint16).astype(jnp.int32)
    hi = lax.bitcast_convert_type(
        t[:, HALF:].astype(jnp.bfloat16), jnp.uint16).astype(jnp.int32)
    out_ref[...] = (hi << 16) | lo


def _pack_table(table, rows: int):
    v = table.shape[0]
    return pl.pallas_call(
        _pack_body,
        grid=(v // rows,),
        in_specs=[pl.BlockSpec((rows, EMBED_DIM), lambda i: (i, 0))],
        out_specs=pl.BlockSpec((rows, HALF), lambda i: (i, 0)),
        out_shape=jax.ShapeDtypeStruct((v, HALF), jnp.int32),
        compiler_params=pltpu.CompilerParams(
            dimension_semantics=("parallel",),
        ),
    )(table)


def _make_sc_gather(n_tokens: int):
    per_w = n_tokens // _NW
    chunks = per_w // _CHUNK

    mesh = plsc.VectorSubcoreMesh(core_axis_name="c", subcore_axis_name="s")

    @functools.partial(
        pl.kernel,
        mesh=mesh,
        out_type=jax.ShapeDtypeStruct((n_tokens, HALF), jnp.int32),
        scratch_types=[
            pltpu.VMEM((chunks, _CHUNK), jnp.int32),
            pltpu.VMEM((_CHUNK, HALF), jnp.int32),
            pltpu.SemaphoreType.DMA,
        ],
        compiler_params=pltpu.CompilerParams(use_tc_tiling_on_sc=False),
    )
    def sc_gather(table_hbm, idx_hbm, out_hbm, idx_v, rows_v, gsem):
        wid = lax.axis_index("s") * _NC + lax.axis_index("c")
        row_base = wid * chunks
        pltpu.sync_copy(idx_hbm.at[pl.ds(row_base, chunks)], idx_v)

        def body(g, carry):
            pltpu.async_copy(table_hbm.at[idx_v.at[g]], rows_v, gsem).wait()
            tok = (row_base + g) * _CHUNK
            pltpu.sync_copy(rows_v, out_hbm.at[pl.ds(tok, _CHUNK)])
            return carry

        lax.fori_loop(0, chunks, body, 0)

    return sc_gather


def _mm_body(e_ref, wlo_ref, whi_ref, out_ref):
    e32 = e_ref[...]
    lo = lax.bitcast_convert_type(e32 << 16, jnp.float32)
    hi = lax.bitcast_convert_type(e32 & jnp.int32(-65536), jnp.float32)
    dims = (((1,), (1,)), ((), ()))
    out_ref[...] = (
        lax.dot_general(lo, wlo_ref[...], dims,
                        preferred_element_type=jnp.float32)
        + lax.dot_general(hi, whi_ref[...], dims,
                          preferred_element_type=jnp.float32)
    )


def _project(emb32, w, tile: int):
    n = emb32.shape[0]
    return pl.pallas_call(
        _mm_body,
        grid=(n // tile,),
        in_specs=[
            pl.BlockSpec((tile, HALF), lambda i: (i, 0)),
            pl.BlockSpec((ATTN_DIM, HALF), lambda i: (0, 0)),
            pl.BlockSpec((ATTN_DIM, HALF), lambda i: (0, 0)),
        ],
        out_specs=pl.BlockSpec((tile, ATTN_DIM), lambda i: (i, 0)),
        out_shape=jax.ShapeDtypeStruct((n, ATTN_DIM), jnp.float32),
        compiler_params=pltpu.CompilerParams(
            dimension_semantics=("parallel",),
        ),
    )(emb32, w[:, :HALF], w[:, HALF:])


def kernel(x, table, W):
    b, s = x.shape
    n = b * s
    idx2d = x.reshape(n // _CHUNK, _CHUNK).astype(jnp.int32)
    tb32 = _pack_table(table, rows=1000)
    emb32 = _make_sc_gather(n)(tb32, idx2d)
    out = _project(emb32, W, tile=1024)
    return out.reshape(b, s, ATTN_DIM)


# speedup vs baseline: 1.3567x; 1.3567x over previous
import functools

import jax
import jax.numpy as jnp
from jax import lax
from jax.experimental import pallas as pl
from jax.experimental.pallas import tpu as pltpu
from jax.experimental.pallas import tpu_sc as plsc

EMBED_DIM = 128
ATTN_DIM = 768

_NC = 2
_NS = 16
_NW = _NC * _NS

_CHUNK = 128
_NBUF = 4


def _make_sc_gather(n_tokens: int):
    per_w = n_tokens // _NW
    chunks = per_w // _CHUNK
    assert chunks % _NBUF == 0

    mesh = plsc.VectorSubcoreMesh(core_axis_name="c", subcore_axis_name="s")

    @functools.partial(
        pl.kernel,
        mesh=mesh,
        out_type=jax.ShapeDtypeStruct((n_tokens, EMBED_DIM), jnp.float32),
        scratch_types=[
            pltpu.VMEM((chunks, _CHUNK), jnp.int32),
            *[pltpu.VMEM((_CHUNK, EMBED_DIM), jnp.float32)] * _NBUF,
            *[pltpu.SemaphoreType.DMA] * (2 * _NBUF),
        ],
    )
    def sc_gather(table_hbm, idx_hbm, out_hbm, idx_v, *bufs_sems):
        rows = bufs_sems[:_NBUF]
        gsem = bufs_sems[_NBUF:2 * _NBUF]
        wsem = bufs_sems[2 * _NBUF:]
        wid = lax.axis_index("s") * _NC + lax.axis_index("c")
        row_base = wid * chunks
        pltpu.sync_copy(idx_hbm.at[pl.ds(row_base, chunks)], idx_v)

        def gather(b, g):
            pltpu.make_async_copy(
                table_hbm.at[idx_v.at[g]], rows[b], gsem[b]).start()

        def put(b, g):
            tok = (row_base + g) * _CHUNK
            pltpu.make_async_copy(
                rows[b], out_hbm.at[pl.ds(tok, _CHUNK)], wsem[b]).start()

        for b in range(_NBUF):
            gather(b, b)

        def body(gg, carry):
            for b in range(_NBUF):
                g = gg * _NBUF + b
                pltpu.make_async_copy(
                    table_hbm.at[idx_v.at[g]], rows[b], gsem[b]).wait()
                put(b, g)
            for b in range(_NBUF):
                g2 = (gg + 1) * _NBUF + b
                tok = (row_base + g2 - _NBUF) * _CHUNK
                pltpu.make_async_copy(
                    rows[b], out_hbm.at[pl.ds(tok, _CHUNK)], wsem[b]).wait()

                @pl.when(g2 < chunks)
                def _():
                    gather(b, g2)
            return carry

        lax.fori_loop(0, chunks // _NBUF, body, 0)

    return sc_gather


def _mm_body(emb_ref, w_ref, out_ref):
    out_ref[...] = lax.dot_general(
        emb_ref[...], w_ref[...],
        dimension_numbers=(((1,), (1,)), ((), ())),
        preferred_element_type=jnp.float32,
    )


def _project(emb, w, tile: int):
    n = emb.shape[0]
    return pl.pallas_call(
        _mm_body,
        grid=(n // tile,),
        in_specs=[
            pl.BlockSpec((tile, EMBED_DIM), lambda i: (i, 0)),
            pl.BlockSpec((ATTN_DIM, EMBED_DIM), lambda i: (0, 0)),
        ],
        out_specs=pl.BlockSpec((tile, ATTN_DIM), lambda i: (i, 0)),
        out_shape=jax.ShapeDtypeStruct((n, ATTN_DIM), jnp.float32),
        compiler_params=pltpu.CompilerParams(
            dimension_semantics=("parallel",),
        ),
    )(emb, w)


def kernel(x, table, W):
    b, s = x.shape
    n = b * s
    idx2d = x.reshape(n // _CHUNK, _CHUNK).astype(jnp.int32)
    emb = _make_sc_gather(n)(table, idx2d)
    out = _project(emb, W, tile=1024)
    return out.reshape(b, s, ATTN_DIM)


# speedup vs baseline: 1.5979x; 1.1777x over previous
import functools

import jax
import jax.numpy as jnp
from jax import lax
from jax.experimental import pallas as pl
from jax.experimental.pallas import tpu as pltpu
from jax.experimental.pallas import tpu_sc as plsc

EMBED_DIM = 128
ATTN_DIM = 768

_NC = 2
_NS = 16
_NW = _NC * _NS

_CHUNK = 128
_NBUF = 4


def _make_sc_gather(n_tokens: int):
    per_w = n_tokens // _NW
    chunks = per_w // _CHUNK
    assert chunks % _NBUF == 0

    mesh = plsc.VectorSubcoreMesh(core_axis_name="c", subcore_axis_name="s")

    @functools.partial(
        pl.kernel,
        mesh=mesh,
        out_type=jax.ShapeDtypeStruct((n_tokens, EMBED_DIM), jnp.float32),
        scratch_types=[
            pltpu.VMEM((chunks, _CHUNK), jnp.int32),
            *[pltpu.VMEM((_CHUNK, EMBED_DIM), jnp.float32)] * _NBUF,
            *[pltpu.SemaphoreType.DMA] * (2 * _NBUF),
        ],
    )
    def sc_gather(table_hbm, idx_hbm, out_hbm, idx_v, *bufs_sems):
        rows = bufs_sems[:_NBUF]
        gsem = bufs_sems[_NBUF:2 * _NBUF]
        wsem = bufs_sems[2 * _NBUF:]
        wid = lax.axis_index("s") * _NC + lax.axis_index("c")
        row_base = wid * chunks
        pltpu.sync_copy(idx_hbm.at[pl.ds(row_base, chunks)], idx_v)

        def gather(b, g):
            pltpu.make_async_copy(
                table_hbm.at[idx_v.at[g]], rows[b], gsem[b]).start()

        def put(b, g):
            tok = (row_base + g) * _CHUNK
            pltpu.make_async_copy(
                rows[b], out_hbm.at[pl.ds(tok, _CHUNK)], wsem[b]).start()

        for b in range(_NBUF):
            gather(b, b)

        def body(gg, carry):
            for b in range(_NBUF):
                g = gg * _NBUF + b
                pltpu.make_async_copy(
                    table_hbm.at[idx_v.at[g]], rows[b], gsem[b]).wait()
                put(b, g)
            for b in range(_NBUF):
                g2 = (gg + 1) * _NBUF + b
                tok = (row_base + g2 - _NBUF) * _CHUNK
                pltpu.make_async_copy(
                    rows[b], out_hbm.at[pl.ds(tok, _CHUNK)], wsem[b]).wait()

                @pl.when(g2 < chunks)
                def _():
                    gather(b, g2)
            return carry

        lax.fori_loop(0, chunks // _NBUF, body, 0)

    return sc_gather


def _mm_body(emb_ref, w_ref, out_ref):
    out_ref[...] = lax.dot_general(
        emb_ref[...], w_ref[...],
        dimension_numbers=(((1,), (1,)), ((), ())),
        preferred_element_type=jnp.float32,
    )


def _project(emb, w, tile: int):
    n = emb.shape[0]
    return pl.pallas_call(
        _mm_body,
        grid=(n // tile,),
        in_specs=[
            pl.BlockSpec((tile, EMBED_DIM), lambda i: (i, 0)),
            pl.BlockSpec((ATTN_DIM, EMBED_DIM), lambda i: (0, 0)),
        ],
        out_specs=pl.BlockSpec((tile, ATTN_DIM), lambda i: (i, 0)),
        out_shape=jax.ShapeDtypeStruct((n, ATTN_DIM), jnp.float32),
        compiler_params=pltpu.CompilerParams(
            dimension_semantics=("parallel",),
        ),
    )(emb, w)


def kernel(x, table, W):
    b, s = x.shape
    n = b * s
    idx2d = x.reshape(n // _CHUNK, _CHUNK).astype(jnp.int32)
    emb = _make_sc_gather(n)(table, idx2d)
    out = _project(emb, W, tile=2048)
    return out.reshape(b, s, ATTN_DIM)


# speedup vs baseline: 1.6447x; 1.0293x over previous
import functools

import jax
import jax.numpy as jnp
from jax import lax
from jax.experimental import pallas as pl
from jax.experimental.pallas import tpu as pltpu
from jax.experimental.pallas import tpu_sc as plsc

EMBED_DIM = 128
ATTN_DIM = 768

_NC = 2
_NS = 16
_NW = _NC * _NS

_CHUNK = 128
_NBUF = 4


def _make_sc_gather(n_tokens: int):
    per_w = n_tokens // _NW
    chunks = per_w // _CHUNK
    assert chunks % _NBUF == 0

    mesh = plsc.VectorSubcoreMesh(core_axis_name="c", subcore_axis_name="s")

    @functools.partial(
        pl.kernel,
        mesh=mesh,
        out_type=jax.ShapeDtypeStruct((n_tokens, EMBED_DIM), jnp.float32),
        scratch_types=[
            pltpu.VMEM((chunks, _CHUNK), jnp.int32),
            *[pltpu.VMEM((_CHUNK, EMBED_DIM), jnp.float32)] * _NBUF,
            *[pltpu.SemaphoreType.DMA] * (2 * _NBUF),
        ],
    )
    def sc_gather(table_hbm, idx_hbm, out_hbm, idx_v, *bufs_sems):
        rows = bufs_sems[:_NBUF]
        gsem = bufs_sems[_NBUF:2 * _NBUF]
        wsem = bufs_sems[2 * _NBUF:]
        wid = lax.axis_index("s") * _NC + lax.axis_index("c")
        row_base = wid * chunks
        pltpu.sync_copy(idx_hbm.at[pl.ds(row_base, chunks)], idx_v)

        def gather(b, g):
            pltpu.make_async_copy(
                table_hbm.at[idx_v.at[g]], rows[b], gsem[b]).start()

        def put(b, g):
            tok = (row_base + g) * _CHUNK
            pltpu.make_async_copy(
                rows[b], out_hbm.at[pl.ds(tok, _CHUNK)], wsem[b]).start()

        for b in range(_NBUF):
            gather(b, b)

        def body(gg, carry):
            for b in range(_NBUF):
                g = gg * _NBUF + b
                pltpu.make_async_copy(
                    table_hbm.at[idx_v.at[g]], rows[b], gsem[b]).wait()
                put(b, g)
            for b in range(_NBUF):
                g2 = (gg + 1) * _NBUF + b
                tok = (row_base + g2 - _NBUF) * _CHUNK
                pltpu.make_async_copy(
                    rows[b], out_hbm.at[pl.ds(tok, _CHUNK)], wsem[b]).wait()

                @pl.when(g2 < chunks)
                def _():
                    gather(b, g2)
            return carry

        lax.fori_loop(0, chunks // _NBUF, body, 0)

    return sc_gather


def _mm_body(emb_ref, w_ref, out_ref):
    out_ref[...] = lax.dot_general(
        emb_ref[...], w_ref[...],
        dimension_numbers=(((1,), (1,)), ((), ())),
        preferred_element_type=jnp.float32,
    )


def _project(emb, w, tile: int):
    n = emb.shape[0]
    return pl.pallas_call(
        _mm_body,
        grid=(n // tile,),
        in_specs=[
            pl.BlockSpec((tile, EMBED_DIM), lambda i: (i, 0)),
            pl.BlockSpec((ATTN_DIM, EMBED_DIM), lambda i: (0, 0)),
        ],
        out_specs=pl.BlockSpec((tile, ATTN_DIM), lambda i: (i, 0)),
        out_shape=jax.ShapeDtypeStruct((n, ATTN_DIM), jnp.float32),
        compiler_params=pltpu.CompilerParams(
            dimension_semantics=("parallel",),
        ),
    )(emb, w)


def kernel(x, table, W):
    b, s = x.shape
    n = b * s
    idx2d = x.reshape(n // _CHUNK, _CHUNK).astype(jnp.int32)
    emb = _make_sc_gather(n)(table, idx2d)
    out = _project(emb, W, tile=4096)
    return out.reshape(b, s, ATTN_DIM)


# speedup vs baseline: 1.6727x; 1.0170x over previous
import functools

import jax
import jax.numpy as jnp
from jax import lax
from jax.experimental import pallas as pl
from jax.experimental.pallas import tpu as pltpu
from jax.experimental.pallas import tpu_sc as plsc

EMBED_DIM = 128
ATTN_DIM = 768

_NC = 2
_NS = 16
_NW = _NC * _NS

_CHUNK = 128
_NBUF = 4


def _make_sc_gather(n_tokens: int):
    per_w = n_tokens // _NW
    chunks = per_w // _CHUNK
    assert chunks % _NBUF == 0

    mesh = plsc.VectorSubcoreMesh(core_axis_name="c", subcore_axis_name="s")

    @functools.partial(
        pl.kernel,
        mesh=mesh,
        out_type=jax.ShapeDtypeStruct((n_tokens, EMBED_DIM), jnp.float32),
        scratch_types=[
            pltpu.VMEM((chunks, _CHUNK), jnp.int32),
            *[pltpu.VMEM((_CHUNK, EMBED_DIM), jnp.float32)] * _NBUF,
            *[pltpu.SemaphoreType.DMA] * (2 * _NBUF),
        ],
    )
    def sc_gather(table_hbm, idx_hbm, out_hbm, idx_v, *bufs_sems):
        rows = bufs_sems[:_NBUF]
        gsem = bufs_sems[_NBUF:2 * _NBUF]
        wsem = bufs_sems[2 * _NBUF:]
        wid = lax.axis_index("s") * _NC + lax.axis_index("c")
        row_base = wid * chunks
        pltpu.sync_copy(idx_hbm.at[pl.ds(row_base, chunks)], idx_v)

        def gather(b, g):
            pltpu.make_async_copy(
                table_hbm.at[idx_v.at[g]], rows[b], gsem[b]).start()

        def put(b, g):
            tok = (row_base + g) * _CHUNK
            pltpu.make_async_copy(
                rows[b], out_hbm.at[pl.ds(tok, _CHUNK)], wsem[b]).start()

        for b in range(_NBUF):
            gather(b, b)

        def body(gg, carry):
            for b in range(_NBUF):
                g = gg * _NBUF + b
                pltpu.make_async_copy(
                    table_hbm.at[idx_v.at[g]], rows[b], gsem[b]).wait()
                put(b, g)
            for b in range(_NBUF):
                g2 = (gg + 1) * _NBUF + b
                tok = (row_base + g2 - _NBUF) * _CHUNK
                pltpu.make_async_copy(
                    rows[b], out_hbm.at[pl.ds(tok, _CHUNK)], wsem[b]).wait()

                @pl.when(g2 < chunks)
                def _():
                    gather(b, g2)
            return carry

        lax.fori_loop(0, chunks // _NBUF, body, 0)

    return sc_gather


def _mm_body(emb_ref, w_ref, out_ref):
    out_ref[...] = lax.dot_general(
        emb_ref[...], w_ref[...],
        dimension_numbers=(((1,), (1,)), ((), ())),
        preferred_element_type=jnp.float32,
    )


def _project(emb, w, tile: int):
    n = emb.shape[0]
    return pl.pallas_call(
        _mm_body,
        grid=(n // tile,),
        in_specs=[
            pl.BlockSpec((tile, EMBED_DIM), lambda i: (i, 0)),
            pl.BlockSpec((ATTN_DIM, EMBED_DIM), lambda i: (0, 0)),
        ],
        out_specs=pl.BlockSpec((tile, ATTN_DIM), lambda i: (i, 0)),
        out_shape=jax.ShapeDtypeStruct((n, ATTN_DIM), jnp.float32),
        compiler_params=pltpu.CompilerParams(
            dimension_semantics=("parallel",),
        ),
    )(emb, w)


def kernel(x, table, W):
    b, s = x.shape
    n = b * s
    idx2d = x.reshape(n // _CHUNK, _CHUNK).astype(jnp.int32)
    emb = _make_sc_gather(n)(table, idx2d)
    out = _project(emb, W, tile=8192)
    return out.reshape(b, s, ATTN_DIM)
